# Initial kernel scaffold; baseline (speedup 1.0000x reference)
import functools

import jax
import jax.numpy as jnp
from jax import lax
from jax.experimental import pallas as pl
from jax.experimental.pallas import tpu as pltpu
from jax.experimental.pallas import tpu_sc as plsc

N = 10000
D_IN = 128
D_HID = 64
D_OUT = 128
NUM_MPS = 5
EPS = 1e-10

N_PAD = 10016
ROWS_PER_TILE = N_PAD // 16
NTILES = 32
CHUNK = 128
CHUNKS_PER_TILE = 80
E_PAD = NTILES * CHUNKS_PER_TILE * CHUNK


def _mm1_body(x_ref, w_ref, o_ref):
    o_ref[...] = lax.dot_general(
        x_ref[...], w_ref[...], (((1,), (1,)), ((), ())),
        preferred_element_type=jnp.float32)


def _mm1(x_pad, W1):
    return pl.pallas_call(
        _mm1_body,
        out_shape=jax.ShapeDtypeStruct((N_PAD, D_HID), jnp.float32),
    )(x_pad, W1)


def _add_body(p_ref, o_ref):
    o_ref[...] = p_ref[0] + p_ref[1]


def _add(P):
    return pl.pallas_call(
        _add_body,
        out_shape=jax.ShapeDtypeStruct((N_PAD, D_HID), jnp.float32),
    )(P)


def _final_body(h_ref, w2_ref, b1_ref, b2_ref, o_ref):
    h = h_ref[:N] + b1_ref[...]
    mean = jnp.mean(h, axis=0)
    var = jnp.mean((h - mean) ** 2, axis=0)
    hn = (h - mean) * lax.rsqrt(var + EPS)
    hr = jnp.maximum(hn, 0.0)
    o_ref[...] = lax.dot_general(
        hr, w2_ref[...], (((1,), (1,)), ((), ())),
        preferred_element_type=jnp.float32) + b2_ref[...]


def _final(h, W2, b1, b2):
    return pl.pallas_call(
        _final_body,
        out_shape=jax.ShapeDtypeStruct((N, D_OUT), jnp.float32),
    )(h, W2, b1.reshape(1, D_HID), b2.reshape(1, D_OUT))


def _prop_body(h_hbm, src_hbm, dst_hbm, zero_hbm, out_hbm,
               src_v, dst_v, rows0, rows1, zbuf, acc, sem0, sem1):
    c = lax.axis_index("c")
    s = lax.axis_index("s")
    w = c * 16 + s
    pltpu.sync_copy(src_hbm.at[w], src_v)
    pltpu.sync_copy(dst_hbm.at[w], dst_v)
    pltpu.sync_copy(zero_hbm.at[pl.ds(s * ROWS_PER_TILE, ROWS_PER_TILE)], zbuf)
    pltpu.sync_copy(zbuf, acc.at[pl.ds(s * ROWS_PER_TILE, ROWS_PER_TILE)])
    plsc.subcore_barrier()

    rows = (rows0, rows1)
    sems = (sem0, sem1)
    pltpu.async_copy(h_hbm.at[src_v.at[0]], rows0, sem0)
    pltpu.async_copy(h_hbm.at[src_v.at[1]], rows1, sem1)

    def body(g, carry):
        for b in range(2):
            j = 2 * g + b
            pltpu.make_async_copy(h_hbm.at[src_v.at[j]], rows[b], sems[b]).wait()
            pltpu.sync_copy(rows[b], acc.at[dst_v.at[j]], add=True)

            @pl.when(j + 2 < CHUNKS_PER_TILE)
            def _():
                pltpu.async_copy(h_hbm.at[src_v.at[j + 2]], rows[b], sems[b])
        return carry

    lax.fori_loop(0, CHUNKS_PER_TILE // 2, body, 0)
    plsc.subcore_barrier()
    pltpu.sync_copy(acc.at[pl.ds(s * ROWS_PER_TILE, ROWS_PER_TILE)],
                    out_hbm.at[c, pl.ds(s * ROWS_PER_TILE, ROWS_PER_TILE)])


_prop = functools.partial(
    pl.kernel,
    out_type=jax.ShapeDtypeStruct((2, N_PAD, D_HID), jnp.float32),
    mesh=plsc.VectorSubcoreMesh(core_axis_name="c", subcore_axis_name="s"),
    scratch_types=[
        pltpu.VMEM((CHUNKS_PER_TILE, CHUNK), jnp.int32),
        pltpu.VMEM((CHUNKS_PER_TILE, CHUNK), jnp.int32),
        pltpu.VMEM((CHUNK, D_HID), jnp.float32),
        pltpu.VMEM((CHUNK, D_HID), jnp.float32),
        pltpu.VMEM((ROWS_PER_TILE, D_HID), jnp.float32),
        pltpu.VMEM_SHARED((N_PAD, D_HID), jnp.float32),
        pltpu.SemaphoreType.DMA,
        pltpu.SemaphoreType.DMA,
    ],
)(_prop_body)


def kernel(x, edge_index, W1, b1, W2, b2):
    dst = edge_index[0]
    src = edge_index[1]
    e = dst.shape[0]
    pad = E_PAD - e
    ar = jnp.arange(pad, dtype=jnp.int32)
    pad_src = (ar * 131) % N
    pad_dst = N + (ar % (N_PAD - N))
    src_p = jnp.concatenate([src, pad_src]).reshape(NTILES, CHUNKS_PER_TILE, CHUNK)
    dst_p = jnp.concatenate([dst, pad_dst]).reshape(NTILES, CHUNKS_PER_TILE, CHUNK)
    x_pad = jnp.pad(x, ((0, N_PAD - N), (0, 0)))
    zeros = jnp.zeros((N_PAD, D_HID), jnp.float32)

    h = _mm1(x_pad, W1)
    for _ in range(NUM_MPS):
        P = _prop(h, src_p, dst_p, zeros)
        h = _add(P)
    return _final(h, W2, b1, b2)


# speedup vs baseline: 8.2617x; 8.2617x over previous
import functools

import jax
import jax.numpy as jnp
from jax import lax
from jax.experimental import pallas as pl
from jax.experimental.pallas import tpu as pltpu
from jax.experimental.pallas import tpu_sc as plsc

N = 10000
D_IN = 128
D_HID = 64
D_OUT = 128
D_HALF = 64
NUM_MPS = 5
EPS = 1e-10

N_PAD = 10112
ROWS_PER_TILE = N_PAD // 16
NTILES = 32
CHUNK = 128
CHUNKS_PER_TILE = 80
E_PAD = NTILES * CHUNKS_PER_TILE * CHUNK


def _add_body(p_ref, o_ref):
    o_ref[...] = p_ref[0] + p_ref[1]


def _add(P):
    return pl.pallas_call(
        _add_body,
        out_shape=jax.ShapeDtypeStruct((2, N_PAD, D_HALF), jnp.float32),
    )(P)


def _final_body(h_ref, w1_ref, b1_ref, w2_ref, b2_ref, o_ref):
    h5 = jnp.concatenate([h_ref[0, :N], h_ref[1, :N]], axis=1)
    h = lax.dot_general(
        h5, w1_ref[...], (((1,), (1,)), ((), ())),
        preferred_element_type=jnp.float32) + b1_ref[...]
    mean = jnp.mean(h, axis=0)
    var = jnp.mean((h - mean) ** 2, axis=0)
    hn = (h - mean) / jnp.sqrt(var + EPS)
    hr = jnp.maximum(hn, 0.0)
    o_ref[...] = lax.dot_general(
        hr, w2_ref[...], (((1,), (1,)), ((), ())),
        preferred_element_type=jnp.float32) + b2_ref[...]


def _final(h, W1, b1, W2, b2):
    return pl.pallas_call(
        _final_body,
        out_shape=jax.ShapeDtypeStruct((N, D_OUT), jnp.float32),
    )(h, W1, b1.reshape(1, D_HID), W2, b2.reshape(1, D_OUT))


def _prop_body(h_hbm, src_hbm, dst_hbm, zero_hbm, out_hbm,
               src_v, dst_v, rows0, rows1, acc, sem0, sem1):
    c = lax.axis_index("c")
    s = lax.axis_index("s")
    w = c * 16 + s
    row_slice = pl.ds(s * ROWS_PER_TILE, ROWS_PER_TILE)
    pltpu.sync_copy(src_hbm.at[w], src_v)
    pltpu.sync_copy(dst_hbm.at[w], dst_v)

    rows = (rows0, rows1)
    sems = (sem0, sem1)

    for q in range(2):
        h_q = h_hbm.at[q]
        pltpu.sync_copy(zero_hbm.at[row_slice], acc.at[row_slice])
        plsc.subcore_barrier()

        pltpu.async_copy(h_q.at[src_v.at[0]], rows0, sem0)
        pltpu.async_copy(h_q.at[src_v.at[1]], rows1, sem1)

        def body(g, carry):
            for b in range(2):
                j = 2 * g + b
                pltpu.make_async_copy(h_q.at[src_v.at[j]], rows[b], sems[b]).wait()
                pltpu.sync_copy(rows[b], acc.at[dst_v.at[j]], add=True)

                @pl.when(j + 2 < CHUNKS_PER_TILE)
                def _():
                    pltpu.async_copy(h_q.at[src_v.at[j + 2]], rows[b], sems[b])
            return carry

        lax.fori_loop(0, CHUNKS_PER_TILE // 2, body, 0)
        plsc.subcore_barrier()
        pltpu.sync_copy(acc.at[row_slice], out_hbm.at[c, q, row_slice])


_prop = functools.partial(
    pl.kernel,
    out_type=jax.ShapeDtypeStruct((2, 2, N_PAD, D_HALF), jnp.float32),
    mesh=plsc.VectorSubcoreMesh(core_axis_name="c", subcore_axis_name="s"),
    compiler_params=pltpu.CompilerParams(use_tc_tiling_on_sc=False),
    scratch_types=[
        pltpu.VMEM((CHUNKS_PER_TILE, CHUNK), jnp.int32),
        pltpu.VMEM((CHUNKS_PER_TILE, CHUNK), jnp.int32),
        pltpu.VMEM((CHUNK, D_HALF), jnp.float32),
        pltpu.VMEM((CHUNK, D_HALF), jnp.float32),
        pltpu.VMEM_SHARED((N_PAD, D_HALF), jnp.float32),
        pltpu.SemaphoreType.DMA,
        pltpu.SemaphoreType.DMA,
    ],
)(_prop_body)


def kernel(x, edge_index, W1, b1, W2, b2):
    dst = edge_index[0]
    src = edge_index[1]
    e = dst.shape[0]
    pad = E_PAD - e
    ar = jnp.arange(pad, dtype=jnp.int32)
    pad_src = (ar * 131) % N
    pad_dst = N + (ar % (N_PAD - N))
    src_p = jnp.concatenate([src, pad_src]).reshape(NTILES, CHUNKS_PER_TILE, CHUNK)
    dst_p = jnp.concatenate([dst, pad_dst]).reshape(NTILES, CHUNKS_PER_TILE, CHUNK)
    x_pad = jnp.pad(x, ((0, N_PAD - N), (0, 0)))
    zeros = jnp.zeros((N_PAD, D_HALF), jnp.float32)

    h = jnp.stack([x_pad[:, :D_HALF], x_pad[:, D_HALF:]])
    for _ in range(NUM_MPS):
        P = _prop(h, src_p, dst_p, zeros)
        h = _add(P)
    return _final(h, W1, b1, W2, b2)


# speedup vs baseline: 9.7089x; 1.1752x over previous
import functools

import jax
import jax.numpy as jnp
from jax import lax
from jax.experimental import pallas as pl
from jax.experimental.pallas import tpu as pltpu
from jax.experimental.pallas import tpu_sc as plsc

N = 10000
D_IN = 128
D_HID = 64
D_OUT = 128
D_HALF = 64
NUM_MPS = 5
EPS = 1e-10

N_PAD = 10112
ROWS_PER_TILE = N_PAD // 16
NTILES = 32
CHUNK = 128
CHUNKS_PER_TILE = 80
E_PAD = NTILES * CHUNKS_PER_TILE * CHUNK


def _add_body(p_ref, o_ref):
    o_ref[...] = p_ref[0] + p_ref[1]


def _add(P):
    return pl.pallas_call(
        _add_body,
        out_shape=jax.ShapeDtypeStruct((2, N_PAD, D_HALF), jnp.float32),
    )(P)


def _final_body(h_ref, w1_ref, b1_ref, w2_ref, b2_ref, o_ref):
    h5 = jnp.concatenate([h_ref[0, :N], h_ref[1, :N]], axis=1)
    h = lax.dot_general(
        h5, w1_ref[...], (((1,), (1,)), ((), ())),
        preferred_element_type=jnp.float32) + b1_ref[...]
    mean = jnp.mean(h, axis=0)
    var = jnp.mean((h - mean) ** 2, axis=0)
    hn = (h - mean) / jnp.sqrt(var + EPS)
    hr = jnp.maximum(hn, 0.0)
    o_ref[...] = lax.dot_general(
        hr, w2_ref[...], (((1,), (1,)), ((), ())),
        preferred_element_type=jnp.float32) + b2_ref[...]


def _final(h, W1, b1, W2, b2):
    return pl.pallas_call(
        _final_body,
        out_shape=jax.ShapeDtypeStruct((N, D_OUT), jnp.float32),
    )(h, W1, b1.reshape(1, D_HID), W2, b2.reshape(1, D_OUT))


NBUF = 8
LOOKAHEAD = 4


def _prop_body(h_hbm, src_hbm, dst_hbm, zero_hbm, out_hbm,
               src_v, dst_v, rows, gsems, ssems, acc):
    c = lax.axis_index("c")
    s = lax.axis_index("s")
    w = c * 16 + s
    row_slice = pl.ds(s * ROWS_PER_TILE, ROWS_PER_TILE)
    pltpu.sync_copy(src_hbm.at[w], src_v)
    pltpu.sync_copy(dst_hbm.at[w], dst_v)

    for q in range(2):
        h_q = h_hbm.at[q]
        pltpu.sync_copy(zero_hbm.at[row_slice], acc.at[row_slice])
        plsc.subcore_barrier()

        for j0 in range(LOOKAHEAD):
            pltpu.async_copy(h_q.at[src_v.at[j0]], rows.at[j0], gsems.at[j0])

        def body(g, carry):
            for b in range(NBUF):
                j = NBUF * g + b
                bg = (b + LOOKAHEAD) % NBUF

                @pl.when(j + LOOKAHEAD < CHUNKS_PER_TILE)
                def _():
                    @pl.when(j + LOOKAHEAD >= NBUF)
                    def _():
                        pltpu.make_async_copy(
                            rows.at[bg], acc.at[dst_v.at[j + LOOKAHEAD - NBUF]],
                            ssems.at[bg]).wait()
                    pltpu.async_copy(
                        h_q.at[src_v.at[j + LOOKAHEAD]], rows.at[bg],
                        gsems.at[bg])

                pltpu.make_async_copy(
                    h_q.at[src_v.at[j]], rows.at[b], gsems.at[b]).wait()
                pltpu.async_copy(rows.at[b], acc.at[dst_v.at[j]], ssems.at[b],
                                 add=True)
            return carry

        lax.fori_loop(0, CHUNKS_PER_TILE // NBUF, body, 0)
        for b in range(NBUF):
            pltpu.make_async_copy(
                rows.at[b],
                acc.at[dst_v.at[CHUNKS_PER_TILE - NBUF + b]],
                ssems.at[b]).wait()
        plsc.subcore_barrier()
        pltpu.sync_copy(acc.at[row_slice], out_hbm.at[c, q, row_slice])


_prop = functools.partial(
    pl.kernel,
    out_type=jax.ShapeDtypeStruct((2, 2, N_PAD, D_HALF), jnp.float32),
    mesh=plsc.VectorSubcoreMesh(core_axis_name="c", subcore_axis_name="s"),
    compiler_params=pltpu.CompilerParams(use_tc_tiling_on_sc=False),
    scratch_types=[
        pltpu.VMEM((CHUNKS_PER_TILE, CHUNK), jnp.int32),
        pltpu.VMEM((CHUNKS_PER_TILE, CHUNK), jnp.int32),
        pltpu.VMEM((NBUF, CHUNK, D_HALF), jnp.float32),
        pltpu.SemaphoreType.DMA((NBUF,)),
        pltpu.SemaphoreType.DMA((NBUF,)),
        pltpu.VMEM_SHARED((N_PAD, D_HALF), jnp.float32),
    ],
)(_prop_body)


def kernel(x, edge_index, W1, b1, W2, b2):
    dst = edge_index[0]
    src = edge_index[1]
    e = dst.shape[0]
    pad = E_PAD - e
    ar = jnp.arange(pad, dtype=jnp.int32)
    pad_src = (ar * 131) % N
    pad_dst = N + (ar % (N_PAD - N))
    src_p = jnp.concatenate([src, pad_src]).reshape(NTILES, CHUNKS_PER_TILE, CHUNK)
    dst_p = jnp.concatenate([dst, pad_dst]).reshape(NTILES, CHUNKS_PER_TILE, CHUNK)
    x_pad = jnp.pad(x, ((0, N_PAD - N), (0, 0)))
    zeros = jnp.zeros((N_PAD, D_HALF), jnp.float32)

    h = jnp.stack([x_pad[:, :D_HALF], x_pad[:, D_HALF:]])
    for _ in range(NUM_MPS):
        P = _prop(h, src_p, dst_p, zeros)
        h = _add(P)
    return _final(h, W1, b1, W2, b2)


# speedup vs baseline: 10.0270x; 1.0328x over previous
import functools

import jax
import jax.numpy as jnp
from jax import lax
from jax.experimental import pallas as pl
from jax.experimental.pallas import tpu as pltpu
from jax.experimental.pallas import tpu_sc as plsc

N = 10000
D_IN = 128
D_HID = 64
D_OUT = 128
D_HALF = 64
NUM_MPS = 5
EPS = 1e-10

N_PAD = 10112
ROWS_PER_TILE = N_PAD // 16
NTILES = 32
CHUNK = 128
CHUNKS_PER_TILE = 80
E_PAD = NTILES * CHUNKS_PER_TILE * CHUNK


def _add_body(p_ref, o_ref):
    o_ref[...] = p_ref[0] + p_ref[1]


def _add(P):
    return pl.pallas_call(
        _add_body,
        out_shape=jax.ShapeDtypeStruct((2, N_PAD, D_HALF), jnp.float32),
    )(P)


def _final_body(h_ref, w1_ref, b1_ref, w2_ref, b2_ref, o_ref):
    h5 = jnp.concatenate([h_ref[0, :N], h_ref[1, :N]], axis=1)
    h = lax.dot_general(
        h5, w1_ref[...], (((1,), (1,)), ((), ())),
        preferred_element_type=jnp.float32) + b1_ref[...]
    mean = jnp.mean(h, axis=0)
    var = jnp.mean((h - mean) ** 2, axis=0)
    hn = (h - mean) / jnp.sqrt(var + EPS)
    hr = jnp.maximum(hn, 0.0)
    o_ref[...] = lax.dot_general(
        hr, w2_ref[...], (((1,), (1,)), ((), ())),
        preferred_element_type=jnp.float32) + b2_ref[...]


def _final(h, W1, b1, W2, b2):
    return pl.pallas_call(
        _final_body,
        out_shape=jax.ShapeDtypeStruct((N, D_OUT), jnp.float32),
    )(h, W1, b1.reshape(1, D_HID), W2, b2.reshape(1, D_OUT))


NBUF = 8
LOOKAHEAD = 5


def _prop_body(h_hbm, src_hbm, dst_hbm, zero_hbm, out_hbm,
               src_v, dst_v, rows, gsems, ssems, acc):
    c = lax.axis_index("c")
    s = lax.axis_index("s")
    w = c * 16 + s
    row_slice = pl.ds(s * ROWS_PER_TILE, ROWS_PER_TILE)
    pltpu.sync_copy(src_hbm.at[w], src_v)
    pltpu.sync_copy(dst_hbm.at[w], dst_v)

    for q in range(2):
        h_q = h_hbm.at[q]
        pltpu.sync_copy(zero_hbm.at[row_slice], acc.at[row_slice])
        plsc.subcore_barrier()

        for j0 in range(LOOKAHEAD):
            pltpu.async_copy(h_q.at[src_v.at[j0]], rows.at[j0], gsems.at[j0])

        def body(g, carry):
            for b in range(NBUF):
                j = NBUF * g + b
                bg = (b + LOOKAHEAD) % NBUF

                @pl.when(j + LOOKAHEAD < CHUNKS_PER_TILE)
                def _():
                    @pl.when(j + LOOKAHEAD >= NBUF)
                    def _():
                        pltpu.make_async_copy(
                            rows.at[bg], acc.at[dst_v.at[j + LOOKAHEAD - NBUF]],
                            ssems.at[bg]).wait()
                    pltpu.async_copy(
                        h_q.at[src_v.at[j + LOOKAHEAD]], rows.at[bg],
                        gsems.at[bg])

                pltpu.make_async_copy(
                    h_q.at[src_v.at[j]], rows.at[b], gsems.at[b]).wait()
                pltpu.async_copy(rows.at[b], acc.at[dst_v.at[j]], ssems.at[b],
                                 add=True)
            return carry

        lax.fori_loop(0, CHUNKS_PER_TILE // NBUF, body, 0)
        for b in range(NBUF):
            pltpu.make_async_copy(
                rows.at[b],
                acc.at[dst_v.at[CHUNKS_PER_TILE - NBUF + b]],
                ssems.at[b]).wait()
        plsc.subcore_barrier()
        pltpu.sync_copy(acc.at[row_slice], out_hbm.at[c, q, row_slice])


_prop = functools.partial(
    pl.kernel,
    out_type=jax.ShapeDtypeStruct((2, 2, N_PAD, D_HALF), jnp.float32),
    mesh=plsc.VectorSubcoreMesh(core_axis_name="c", subcore_axis_name="s"),
    compiler_params=pltpu.CompilerParams(use_tc_tiling_on_sc=False),
    scratch_types=[
        pltpu.VMEM((CHUNKS_PER_TILE, CHUNK), jnp.int32),
        pltpu.VMEM((CHUNKS_PER_TILE, CHUNK), jnp.int32),
        pltpu.VMEM((NBUF, CHUNK, D_HALF), jnp.float32),
        pltpu.SemaphoreType.DMA((NBUF,)),
        pltpu.SemaphoreType.DMA((NBUF,)),
        pltpu.VMEM_SHARED((N_PAD, D_HALF), jnp.float32),
    ],
)(_prop_body)


def kernel(x, edge_index, W1, b1, W2, b2):
    dst = edge_index[0]
    src = edge_index[1]
    e = dst.shape[0]
    pad = E_PAD - e
    ar = jnp.arange(pad, dtype=jnp.int32)
    pad_src = (ar * 131) % N
    pad_dst = N + (ar % (N_PAD - N))
    src_p = jnp.concatenate([src, pad_src]).reshape(NTILES, CHUNKS_PER_TILE, CHUNK)
    dst_p = jnp.concatenate([dst, pad_dst]).reshape(NTILES, CHUNKS_PER_TILE, CHUNK)
    x_pad = jnp.pad(x, ((0, N_PAD - N), (0, 0)))
    zeros = jnp.zeros((N_PAD, D_HALF), jnp.float32)

    h = jnp.stack([x_pad[:, :D_HALF], x_pad[:, D_HALF:]])
    for _ in range(NUM_MPS):
        P = _prop(h, src_p, dst_p, zeros)
        h = _add(P)
    return _final(h, W1, b1, W2, b2)


# speedup vs baseline: 11.2239x; 1.1194x over previous
import functools

import jax
import jax.numpy as jnp
from jax import lax
from jax.experimental import pallas as pl
from jax.experimental.pallas import tpu as pltpu
from jax.experimental.pallas import tpu_sc as plsc

N = 10000
D_IN = 128
D_HID = 64
D_OUT = 128
D_HALF = 64
NUM_MPS = 5
EPS = 1e-10

N_PAD = 10112
ROWS_PER_TILE = N_PAD // 16
HALF_ROWS = N_PAD // 2
CROWS = HALF_ROWS // 16
NTILES = 32
CHUNK = 128
CHUNKS_PER_TILE = 80
E_PAD = NTILES * CHUNKS_PER_TILE * CHUNK


def _final_body(h_ref, w1_ref, b1_ref, w2_ref, b2_ref, o_ref):
    h5 = jnp.concatenate([h_ref[0, :N], h_ref[1, :N]], axis=1)
    h = lax.dot_general(
        h5, w1_ref[...], (((1,), (1,)), ((), ())),
        preferred_element_type=jnp.float32) + b1_ref[...]
    mean = jnp.mean(h, axis=0)
    var = jnp.mean((h - mean) ** 2, axis=0)
    hn = (h - mean) / jnp.sqrt(var + EPS)
    hr = jnp.maximum(hn, 0.0)
    o_ref[...] = lax.dot_general(
        hr, w2_ref[...], (((1,), (1,)), ((), ())),
        preferred_element_type=jnp.float32) + b2_ref[...]


def _final(h, W1, b1, W2, b2):
    return pl.pallas_call(
        _final_body,
        out_shape=jax.ShapeDtypeStruct((N, D_OUT), jnp.float32),
    )(h, W1, b1.reshape(1, D_HID), W2, b2.reshape(1, D_OUT))


NBUF = 8
LOOKAHEAD = 5


def _prop5_body(h0_hbm, src_hbm, dst_hbm, zero_hbm,
                hball_hbm, xbuf_hbm,
                src_v, dst_v, rows, gsems, ssems, acc, csem):
    c = lax.axis_index("c")
    s = lax.axis_index("s")
    w = c * 16 + s
    row_slice = pl.ds(s * ROWS_PER_TILE, ROWS_PER_TILE)
    own_lo = c * HALF_ROWS + s * CROWS
    exp_lo = (1 - c) * HALF_ROWS + s * CROWS
    pltpu.sync_copy(src_hbm.at[w], src_v)
    pltpu.sync_copy(dst_hbm.at[w], dst_v)
    stage = rows.at[0].at[pl.ds(0, 79)]
    for qq in range(2):
        for hh in range(8):
            piece = pl.ds(s * ROWS_PER_TILE + hh * 79, 79)
            pltpu.sync_copy(h0_hbm.at[qq, piece], stage)
            pltpu.sync_copy(stage, hball_hbm.at[0, qq, piece])
    plsc.subcore_barrier()
    pltpu.core_barrier(csem, core_axis_name="c")

    def round_body(r, carry):
        in_slot = jnp.where(r == 0, 0, 1 + ((r + 1) % 2))
        out_slot = 1 + (r % 2)

        def pass_body(q, carry_q):
            h_q = hball_hbm.at[in_slot, q]
            pltpu.sync_copy(zero_hbm.at[row_slice], acc.at[row_slice])
            plsc.subcore_barrier()

            for j0 in range(LOOKAHEAD):
                pltpu.async_copy(h_q.at[src_v.at[j0]], rows.at[j0],
                                 gsems.at[j0])

            def body(g, carry_g):
                for b in range(NBUF):
                    j = NBUF * g + b
                    bg = (b + LOOKAHEAD) % NBUF

                    @pl.when(j + LOOKAHEAD < CHUNKS_PER_TILE)
                    def _():
                        @pl.when(j + LOOKAHEAD >= NBUF)
                        def _():
                            pltpu.make_async_copy(
                                rows.at[bg],
                                acc.at[dst_v.at[j + LOOKAHEAD - NBUF]],
                                ssems.at[bg]).wait()
                        pltpu.async_copy(
                            h_q.at[src_v.at[j + LOOKAHEAD]], rows.at[bg],
                            gsems.at[bg])

                    pltpu.make_async_copy(
                        h_q.at[src_v.at[j]], rows.at[b], gsems.at[b]).wait()
                    pltpu.async_copy(rows.at[b], acc.at[dst_v.at[j]],
                                     ssems.at[b], add=True)
                return carry_g

            lax.fori_loop(0, CHUNKS_PER_TILE // NBUF, body, 0)
            for b in range(NBUF):
                pltpu.make_async_copy(
                    rows.at[b],
                    acc.at[dst_v.at[CHUNKS_PER_TILE - NBUF + b]],
                    ssems.at[b]).wait()
            plsc.subcore_barrier()

            pltpu.sync_copy(acc.at[pl.ds(exp_lo, CROWS)],
                            xbuf_hbm.at[c, pl.ds(s * CROWS, CROWS)])
            plsc.subcore_barrier()
            pltpu.core_barrier(csem, core_axis_name="c")

            mine = rows.at[0].at[pl.ds(0, 79)]
            peer = rows.at[1].at[pl.ds(0, 79)]
            for p in range(4):
                pltpu.sync_copy(acc.at[pl.ds(own_lo + p * 79, 79)], mine)
                pltpu.sync_copy(
                    xbuf_hbm.at[1 - c, pl.ds(s * CROWS + p * 79, 79)], peer)

                def add_body(i, carry_a):
                    for k in range(4):
                        col = pl.ds(k * 16, 16)
                        mine[i, col] = mine[i, col] + peer[i, col]
                    return carry_a

                lax.fori_loop(0, 79, add_body, 0)
                pltpu.sync_copy(
                    mine, hball_hbm.at[out_slot, q,
                                       pl.ds(own_lo + p * 79, 79)])
            plsc.subcore_barrier()
            pltpu.core_barrier(csem, core_axis_name="c")
            return carry_q

        lax.fori_loop(0, 2, pass_body, 0)
        return carry

    lax.fori_loop(0, NUM_MPS, round_body, 0)


_prop5 = functools.partial(
    pl.kernel,
    out_type=(
        jax.ShapeDtypeStruct((3, 2, N_PAD, D_HALF), jnp.float32),
        jax.ShapeDtypeStruct((2, HALF_ROWS, D_HALF), jnp.float32),
    ),
    mesh=plsc.VectorSubcoreMesh(core_axis_name="c", subcore_axis_name="s"),
    compiler_params=pltpu.CompilerParams(use_tc_tiling_on_sc=False),
    scratch_types=[
        pltpu.VMEM((CHUNKS_PER_TILE, CHUNK), jnp.int32),
        pltpu.VMEM((CHUNKS_PER_TILE, CHUNK), jnp.int32),
        pltpu.VMEM((NBUF, CHUNK, D_HALF), jnp.float32),
        pltpu.SemaphoreType.DMA((NBUF,)),
        pltpu.SemaphoreType.DMA((NBUF,)),
        pltpu.VMEM_SHARED((N_PAD, D_HALF), jnp.float32),
        pltpu.SemaphoreType.REGULAR,
    ],
)(_prop5_body)


def kernel(x, edge_index, W1, b1, W2, b2):
    dst = edge_index[0]
    src = edge_index[1]
    e = dst.shape[0]
    pad = E_PAD - e
    ar = jnp.arange(pad, dtype=jnp.int32)
    pad_src = (ar * 131) % N
    pad_dst = N + (ar % (N_PAD - N))
    src_p = jnp.concatenate([src, pad_src]).reshape(NTILES, CHUNKS_PER_TILE, CHUNK)
    dst_p = jnp.concatenate([dst, pad_dst]).reshape(NTILES, CHUNKS_PER_TILE, CHUNK)
    x_pad = jnp.pad(x, ((0, N_PAD - N), (0, 0)))
    zeros = jnp.zeros((N_PAD, D_HALF), jnp.float32)

    h0 = jnp.stack([x_pad[:, :D_HALF], x_pad[:, D_HALF:]])
    hball, _ = _prop5(h0, src_p, dst_p, zeros)
    return _final(hball[1], W1, b1, W2, b2)
